# baseline (speedup 1.0000x reference)
import jax
import jax.numpy as jnp
from jax import lax
from jax.experimental import pallas as pl
from jax.experimental.pallas import tpu as pltpu
from jax.experimental.pallas import tpu_sc as plsc

B = 4096
D = 4096
NG = 4
MAXGS = 256
GROUP_SIZES_K = (256, 256, 256, 192)

NC, NS = 2, 16
NW = NC * NS
TOK_PER_W = B // NW
CH = 8
NCH = TOK_PER_W // CH
NBUF = 3

TILE = 256
NT = B // TILE


def _mesh():
    return plsc.VectorSubcoreMesh(core_axis_name="c", subcore_axis_name="s")


def _sc_perm16_body(src2, dest2, perm16_out, src_v, dst_v, bbuf, sem):
    wid = lax.axis_index("s") * NC + lax.axis_index("c")
    pltpu.sync_copy(src2.at[wid], src_v)
    pltpu.sync_copy(dest2.at[wid], dst_v)
    lane = lax.broadcasted_iota(jnp.int32, (16,), 0)
    for c in range(TOK_PER_W // 16):
        vec = src_v[pl.ds(c * 16, 16)]
        for j in range(16):
            s = jnp.max(jnp.where(lane == j, vec, jnp.int32(-1)))
            bbuf[c * 16 + j, pl.ds(0, 16)] = jnp.full((16,), s, jnp.int32)
    for c in range(TOK_PER_W // 16):
        pltpu.async_copy(bbuf.at[pl.ds(c * 16, 16)], perm16_out.at[dst_v.at[c]],
                         sem).wait()


def _sc_gather_pack_body(hid3, srcs2, hp_out, idx_v, fbuf0, fbuf1,
                         pbuf0, pbuf1, sem_g, sem_s):
    wid = lax.axis_index("s") * NC + lax.axis_index("c")
    pltpu.sync_copy(srcs2.at[wid], idx_v)
    fbufs = (fbuf0, fbuf1)
    pbufs = (pbuf0, pbuf1)
    lane = lax.broadcasted_iota(jnp.int32, (16,), 0)

    def start_batch(c, buf):
        vec = idx_v[pl.ds((c // 2) * 16, 16)]
        cps = []
        for j in range(CH):
            l = (c % 2) * CH + j
            s = jnp.max(jnp.where(lane == l, vec, jnp.int32(-1)))
            tok = lax.shift_right_logical(s, 2)
            grp = s - lax.shift_left(tok, 2)
            cps.append(pltpu.async_copy(hid3.at[tok, grp], buf.at[j], sem_g))
        return cps

    def pack_batch(fbuf, pbuf):
        def body(h, carry):
            for q in range(CH // 2):
                a = fbuf[2 * q, pl.ds(h * 16, 16)]
                b = fbuf[2 * q + 1, pl.ds(h * 16, 16)]
                w = plsc.bitcast(
                    plsc.pack(a, b, format=plsc.PackFormat.INTERLEAVED),
                    jnp.float32)
                pbuf[q, pl.ds(h * 16, 16)] = w
            return carry
        lax.fori_loop(0, D // 16, body, 0)

    cps = start_batch(0, fbufs[0])
    wcps = [None, None]
    for c in range(NCH):
        cur = fbufs[c % 2]
        pb = pbufs[c % 2]
        for cp in cps:
            cp.wait()
        if c + 1 < NCH:
            cps = start_batch(c + 1, fbufs[(c + 1) % 2])
        if wcps[c % 2] is not None:
            wcps[c % 2].wait()
        pack_batch(cur, pb)
        wcps[c % 2] = pltpu.async_copy(
            pb, hp_out.at[pl.ds(wid * (TOK_PER_W // 2) + c * (CH // 2),
                                CH // 2)], sem_s)
    for wcp in wcps:
        wcp.wait()


def _sc_unpermute_body(psorted, dest, preds_out, didx_v, pbuf0, pbuf1,
                       sem1, sem2):
    wid = lax.axis_index("s") * NC + lax.axis_index("c")
    base = wid * TOK_PER_W
    half = TOK_PER_W // 2
    pltpu.sync_copy(dest.at[pl.ds(base, TOK_PER_W)], didx_v)
    cp1 = pltpu.async_copy(psorted.at[didx_v.at[pl.ds(0, half)]], pbuf0, sem1)
    cp2 = pltpu.async_copy(psorted.at[didx_v.at[pl.ds(half, half)]], pbuf1,
                           sem2)
    cp1.wait()
    pltpu.sync_copy(pbuf0, preds_out.at[pl.ds(base, half)])
    cp2.wait()
    pltpu.sync_copy(pbuf1, preds_out.at[pl.ds(base + half, half)])


def _tc_gmm_body(h_ref, gs_ref, go_ref, w0, w1, w2, w3, b0, b1, b2, b3,
                 out_ref, valid_ref):
    gs_row = gs_ref[0, 0, :]
    gmin = jnp.min(gs_row)
    gmax = jnp.max(gs_row)
    hb = pltpu.bitcast(h_ref[...], jnp.bfloat16)
    out_ref[...] = jnp.zeros_like(out_ref)
    for gi, (wr, br) in enumerate(zip((w0, w1, w2, w3), (b0, b1, b2, b3))):
        @pl.when((gi >= gmin) & (gi <= gmax))
        def _(wr=wr, br=br, gi=gi):
            pmat = lax.dot_general(hb, wr[...], (((1,), (1,)), ((), ())),
                                   preferred_element_type=jnp.float32)
            pmat = pmat + br[...]
            out_ref[...] = jnp.where(gs_row[:, None] == gi, pmat, out_ref[...])
    go_row = go_ref[0, 0, :]
    gsz = jnp.full_like(go_row, GROUP_SIZES_K[0])
    for kk in range(1, NG):
        gsz = jnp.where(go_row == kk, GROUP_SIZES_K[kk], gsz)
    cols = lax.broadcasted_iota(jnp.int32, (TILE, MAXGS), 1)
    valid_ref[...] = (cols < gsz[:, None]).astype(jnp.int8)


def _sc_perm16(src2, dest2):
    fn = pl.kernel(
        _sc_perm16_body,
        out_type=jax.ShapeDtypeStruct((B, 128), jnp.int32),
        mesh=_mesh(),
        compiler_params=pltpu.CompilerParams(needs_layout_passes=False),
        scratch_types=[
            pltpu.VMEM((TOK_PER_W,), jnp.int32),
            pltpu.VMEM((TOK_PER_W // 16, 16), jnp.int32),
            pltpu.VMEM((TOK_PER_W, 128), jnp.int32),
            pltpu.SemaphoreType.DMA,
        ],
    )
    return fn(src2, dest2)


def _sc_gather_pack(hid3, srcs2):
    fn = pl.kernel(
        _sc_gather_pack_body,
        out_type=jax.ShapeDtypeStruct((B // 2, D), jnp.float32),
        mesh=_mesh(),
        compiler_params=pltpu.CompilerParams(needs_layout_passes=False),
        scratch_types=[
            pltpu.VMEM((TOK_PER_W,), jnp.int32),
            pltpu.VMEM((CH, D), jnp.float32),
            pltpu.VMEM((CH, D), jnp.float32),
            pltpu.VMEM((CH // 2, D), jnp.float32),
            pltpu.VMEM((CH // 2, D), jnp.float32),
            pltpu.SemaphoreType.DMA,
            pltpu.SemaphoreType.DMA,
        ],
    )
    return fn(hid3, srcs2)


def _sc_unpermute(psorted, dest):
    fn = pl.kernel(
        _sc_unpermute_body,
        out_type=jax.ShapeDtypeStruct((B, MAXGS), jnp.float32),
        mesh=_mesh(),
        scratch_types=[
            pltpu.VMEM((TOK_PER_W,), jnp.int32),
            pltpu.VMEM((TOK_PER_W // 2, MAXGS), jnp.float32),
            pltpu.VMEM((TOK_PER_W // 2, MAXGS), jnp.float32),
            pltpu.SemaphoreType.DMA,
            pltpu.SemaphoreType.DMA,
        ],
    )
    return fn(psorted, dest)


def _tc_gmm(h_sorted, g_sorted3, g_orig3, ws, bs):
    return pl.pallas_call(
        _tc_gmm_body,
        grid=(NT,),
        in_specs=[
            pl.BlockSpec((TILE // 2, D), lambda t: (t, 0)),
            pl.BlockSpec((1, 1, TILE), lambda t: (t, 0, 0)),
            pl.BlockSpec((1, 1, TILE), lambda t: (t, 0, 0)),
            pl.BlockSpec((MAXGS, D), lambda t: (0, 0)),
            pl.BlockSpec((MAXGS, D), lambda t: (0, 0)),
            pl.BlockSpec((MAXGS, D), lambda t: (0, 0)),
            pl.BlockSpec((MAXGS, D), lambda t: (0, 0)),
            pl.BlockSpec((1, MAXGS), lambda t: (0, 0)),
            pl.BlockSpec((1, MAXGS), lambda t: (0, 0)),
            pl.BlockSpec((1, MAXGS), lambda t: (0, 0)),
            pl.BlockSpec((1, MAXGS), lambda t: (0, 0)),
        ],
        out_specs=(pl.BlockSpec((TILE, MAXGS), lambda t: (t, 0)),
                   pl.BlockSpec((TILE, MAXGS), lambda t: (t, 0))),
        out_shape=(jax.ShapeDtypeStruct((B, MAXGS), jnp.float32),
                   jax.ShapeDtypeStruct((B, MAXGS), jnp.int8)),
    )(h_sorted, g_sorted3, g_orig3, *ws, *bs)


def kernel(hidden, chosen_group_idx, W0, W1, W2, W3, b0, b1, b2, b3):
    g = chosen_group_idx.astype(jnp.int32)

    onehot = (g[:, None] == jnp.arange(NG, dtype=jnp.int32)[None, :])
    onehot = onehot.astype(jnp.int32)
    counts = jnp.sum(onehot, axis=0)
    offsets = jnp.cumsum(counts) - counts
    rank = jnp.cumsum(onehot, axis=0) - onehot
    dest = jnp.sum((rank + offsets[None, :]) * onehot, axis=1)
    src_row = NG * jnp.arange(B, dtype=jnp.int32) + g

    p = jnp.arange(B, dtype=jnp.int32)
    g_sorted = ((p >= offsets[1]).astype(jnp.int32) +
                (p >= offsets[2]).astype(jnp.int32) +
                (p >= offsets[3]).astype(jnp.int32))

    perm16 = _sc_perm16(src_row.reshape(NW, TOK_PER_W),
                        dest.reshape(NW, TOK_PER_W // 16, 16))
    src_sorted = perm16[:, 0]
    h_sorted = _sc_gather_pack(hidden, src_sorted.reshape(NW, TOK_PER_W))

    ws = (W0.astype(jnp.bfloat16),
          W1.astype(jnp.bfloat16),
          W2.astype(jnp.bfloat16),
          jnp.pad(W3.astype(jnp.bfloat16), ((0, MAXGS - GROUP_SIZES_K[3]),
                                            (0, 0))))
    bs = (b0.reshape(1, MAXGS), b1.reshape(1, MAXGS), b2.reshape(1, MAXGS),
          jnp.pad(b3, (0, MAXGS - GROUP_SIZES_K[3])).reshape(1, MAXGS))
    psorted, valid_i8 = _tc_gmm(h_sorted, g_sorted.reshape(NT, 1, TILE),
                                g.reshape(NT, 1, TILE), ws, bs)

    preds = _sc_unpermute(psorted, dest)
    return preds, valid_i8.astype(jnp.bool_)


# speedup vs baseline: 1.3388x; 1.3388x over previous
import jax
import jax.numpy as jnp
from jax import lax
from jax.experimental import pallas as pl
from jax.experimental.pallas import tpu as pltpu
from jax.experimental.pallas import tpu_sc as plsc

B = 4096
D = 4096
NG = 4
MAXGS = 256
GROUP_SIZES_K = (256, 256, 256, 192)

NC, NS = 2, 16
NW = NC * NS
TOK_PER_W = B // NW
K = 2
SUB = B // K
TPW = SUB // NW
CH = 8
NCH = TPW // CH

TILE = 256
NT = B // TILE
NTK = NT // K


def _mesh():
    return plsc.VectorSubcoreMesh(core_axis_name="c", subcore_axis_name="s")


def _sc_perm16_body(src2, dest2, perm16_out, src_v, dst_v, bbuf, sem):
    wid = lax.axis_index("s") * NC + lax.axis_index("c")
    pltpu.sync_copy(src2.at[wid], src_v)
    pltpu.sync_copy(dest2.at[wid], dst_v)
    lane = lax.broadcasted_iota(jnp.int32, (16,), 0)
    for c in range(TOK_PER_W // 16):
        vec = src_v[pl.ds(c * 16, 16)]
        for j in range(16):
            s = jnp.max(jnp.where(lane == j, vec, jnp.int32(-1)))
            bbuf[c * 16 + j, pl.ds(0, 16)] = jnp.full((16,), s, jnp.int32)
    for c in range(TOK_PER_W // 16):
        pltpu.async_copy(bbuf.at[pl.ds(c * 16, 16)], perm16_out.at[dst_v.at[c]],
                         sem).wait()


def _sc_gather_body(hid3, srcs2, h_out, idx_v, fbuf0, fbuf1, sem_g, sem_s):
    wid = lax.axis_index("s") * NC + lax.axis_index("c")
    pltpu.sync_copy(srcs2.at[wid], idx_v)
    fbufs = (fbuf0, fbuf1)
    lane = lax.broadcasted_iota(jnp.int32, (16,), 0)

    def start_batch(c, buf):
        vec = idx_v[pl.ds((c // 2) * 16, 16)]
        cps = []
        for j in range(CH):
            l = (c % 2) * CH + j
            s = jnp.max(jnp.where(lane == l, vec, jnp.int32(-1)))
            tok = lax.shift_right_logical(s, 2)
            grp = s - lax.shift_left(tok, 2)
            cps.append(pltpu.async_copy(hid3.at[tok, grp], buf.at[j], sem_g))
        return cps

    cps = start_batch(0, fbuf0)
    for c in range(NCH):
        cur = fbufs[c % 2]
        for cp in cps:
            cp.wait()
        if c + 1 < NCH:
            cps = start_batch(c + 1, fbufs[(c + 1) % 2])
        pltpu.async_copy(cur, h_out.at[pl.ds(wid * TPW + c * CH, CH)],
                         sem_s).wait()


def _sc_unpermute_body(psorted, dest, preds_out, didx_v, pbuf0, pbuf1,
                       sem1, sem2):
    wid = lax.axis_index("s") * NC + lax.axis_index("c")
    base = wid * TOK_PER_W
    half = TOK_PER_W // 2
    pltpu.sync_copy(dest.at[pl.ds(base, TOK_PER_W)], didx_v)
    cp1 = pltpu.async_copy(psorted.at[didx_v.at[pl.ds(0, half)]], pbuf0, sem1)
    cp2 = pltpu.async_copy(psorted.at[didx_v.at[pl.ds(half, half)]], pbuf1,
                           sem2)
    cp1.wait()
    pltpu.sync_copy(pbuf0, preds_out.at[pl.ds(base, half)])
    cp2.wait()
    pltpu.sync_copy(pbuf1, preds_out.at[pl.ds(base + half, half)])


def _tc_gmm_body(h_ref, gs_ref, go_ref, w0, w1, w2, w3, b0, b1, b2, b3,
                 pprev, vprev, out_ref, valid_ref):
    del pprev, vprev
    gs_row = gs_ref[0, 0, :]
    gmin = jnp.min(gs_row)
    gmax = jnp.max(gs_row)
    hb = h_ref[...].astype(jnp.bfloat16)
    out_ref[...] = jnp.zeros_like(out_ref)
    for gi, (wr, br) in enumerate(zip((w0, w1, w2, w3), (b0, b1, b2, b3))):
        @pl.when((gi >= gmin) & (gi <= gmax))
        def _(wr=wr, br=br, gi=gi):
            pmat = lax.dot_general(hb, wr[...], (((1,), (1,)), ((), ())),
                                   preferred_element_type=jnp.float32)
            pmat = pmat + br[...]
            out_ref[...] = jnp.where(gs_row[:, None] == gi, pmat, out_ref[...])
    go_row = go_ref[0, 0, :]
    gsz = jnp.full_like(go_row, GROUP_SIZES_K[0])
    for kk in range(1, NG):
        gsz = jnp.where(go_row == kk, GROUP_SIZES_K[kk], gsz)
    cols = lax.broadcasted_iota(jnp.int32, (TILE, MAXGS), 1)
    valid_ref[...] = (cols < gsz[:, None]).astype(jnp.int8)


def _sc_perm16(src2, dest2):
    fn = pl.kernel(
        _sc_perm16_body,
        out_type=jax.ShapeDtypeStruct((B, 128), jnp.int32),
        mesh=_mesh(),
        compiler_params=pltpu.CompilerParams(needs_layout_passes=False),
        scratch_types=[
            pltpu.VMEM((TOK_PER_W,), jnp.int32),
            pltpu.VMEM((TOK_PER_W // 16, 16), jnp.int32),
            pltpu.VMEM((TOK_PER_W, 128), jnp.int32),
            pltpu.SemaphoreType.DMA,
        ],
    )
    return fn(src2, dest2)


def _sc_gather(hid3, srcs2, k):
    fn = pl.kernel(
        _sc_gather_body,
        out_type=jax.ShapeDtypeStruct((SUB, D), jnp.float32),
        mesh=_mesh(),
        compiler_params=pltpu.CompilerParams(needs_layout_passes=False),
        scratch_types=[
            pltpu.VMEM((TPW,), jnp.int32),
            pltpu.VMEM((CH, D), jnp.float32),
            pltpu.VMEM((CH, D), jnp.float32),
            pltpu.SemaphoreType.DMA,
            pltpu.SemaphoreType.DMA,
        ],
        name=f"sc_gather_{k}",
    )
    return fn(hid3, srcs2)


def _sc_unpermute(psorted, dest):
    fn = pl.kernel(
        _sc_unpermute_body,
        out_type=jax.ShapeDtypeStruct((B, MAXGS), jnp.float32),
        mesh=_mesh(),
        scratch_types=[
            pltpu.VMEM((TOK_PER_W,), jnp.int32),
            pltpu.VMEM((TOK_PER_W // 2, MAXGS), jnp.float32),
            pltpu.VMEM((TOK_PER_W // 2, MAXGS), jnp.float32),
            pltpu.SemaphoreType.DMA,
            pltpu.SemaphoreType.DMA,
        ],
    )
    return fn(psorted, dest)


_GMM_IN_SPECS = [
    pl.BlockSpec((TILE, D), lambda t: (t, 0)),
    pl.BlockSpec((1, 1, TILE), lambda t: (t, 0, 0)),
    pl.BlockSpec((1, 1, TILE), lambda t: (t, 0, 0)),
    pl.BlockSpec((MAXGS, D), lambda t: (0, 0)),
    pl.BlockSpec((MAXGS, D), lambda t: (0, 0)),
    pl.BlockSpec((MAXGS, D), lambda t: (0, 0)),
    pl.BlockSpec((MAXGS, D), lambda t: (0, 0)),
    pl.BlockSpec((1, MAXGS), lambda t: (0, 0)),
    pl.BlockSpec((1, MAXGS), lambda t: (0, 0)),
    pl.BlockSpec((1, MAXGS), lambda t: (0, 0)),
    pl.BlockSpec((1, MAXGS), lambda t: (0, 0)),
    pl.BlockSpec(memory_space=pl.ANY),
    pl.BlockSpec(memory_space=pl.ANY),
]


def _tc_gmm_chunk(k, h_k, gs3_k, go3_k, ws, bs, pprev, vprev, aliased):
    return pl.pallas_call(
        _tc_gmm_body,
        grid=(NTK,),
        in_specs=_GMM_IN_SPECS,
        out_specs=(pl.BlockSpec((TILE, MAXGS), lambda t, k=k: (t + NTK * k, 0)),
                   pl.BlockSpec((TILE, MAXGS), lambda t, k=k: (t + NTK * k, 0))),
        out_shape=(jax.ShapeDtypeStruct((B, MAXGS), jnp.float32),
                   jax.ShapeDtypeStruct((B, MAXGS), jnp.int8)),
        input_output_aliases={11: 0, 12: 1} if aliased else {},
        name=f"tc_gmm_{k}",
    )(h_k, gs3_k, go3_k, *ws, *bs, pprev, vprev)


def kernel(hidden, chosen_group_idx, W0, W1, W2, W3, b0, b1, b2, b3):
    g = chosen_group_idx.astype(jnp.int32)

    onehot = (g[:, None] == jnp.arange(NG, dtype=jnp.int32)[None, :])
    onehot = onehot.astype(jnp.int32)
    counts = jnp.sum(onehot, axis=0)
    offsets = jnp.cumsum(counts) - counts
    rank = jnp.cumsum(onehot, axis=0) - onehot
    dest = jnp.sum((rank + offsets[None, :]) * onehot, axis=1)
    src_row = NG * jnp.arange(B, dtype=jnp.int32) + g

    p = jnp.arange(B, dtype=jnp.int32)
    g_sorted = ((p >= offsets[1]).astype(jnp.int32) +
                (p >= offsets[2]).astype(jnp.int32) +
                (p >= offsets[3]).astype(jnp.int32))

    perm16 = _sc_perm16(src_row.reshape(NW, TOK_PER_W),
                        dest.reshape(NW, TOK_PER_W // 16, 16))
    src_sorted = perm16[:, 0]

    ws = (W0.astype(jnp.bfloat16),
          W1.astype(jnp.bfloat16),
          W2.astype(jnp.bfloat16),
          jnp.pad(W3.astype(jnp.bfloat16), ((0, MAXGS - GROUP_SIZES_K[3]),
                                            (0, 0))))
    bs = (b0.reshape(1, MAXGS), b1.reshape(1, MAXGS), b2.reshape(1, MAXGS),
          jnp.pad(b3, (0, MAXGS - GROUP_SIZES_K[3])).reshape(1, MAXGS))

    srck = src_sorted.reshape(K, NW, TPW)
    gs4 = g_sorted.reshape(K, NTK, 1, TILE)
    go4 = g.reshape(K, NTK, 1, TILE)

    pprev = jnp.zeros((8, MAXGS), jnp.float32)
    vprev = jnp.zeros((8, MAXGS), jnp.int8)
    psorted, valid_i8 = pprev, vprev
    for k in range(K):
        h_k = _sc_gather(hidden, srck[k], k)
        psorted, valid_i8 = _tc_gmm_chunk(k, h_k, gs4[k], go4[k], ws, bs,
                                          psorted, valid_i8, aliased=(k > 0))

    preds = _sc_unpermute(psorted, dest)
    return preds, valid_i8.astype(jnp.bool_)


# speedup vs baseline: 1.5181x; 1.1339x over previous
import jax
import jax.numpy as jnp
from jax import lax
from jax.experimental import pallas as pl
from jax.experimental.pallas import tpu as pltpu
from jax.experimental.pallas import tpu_sc as plsc

B = 4096
D = 4096
NG = 4
MAXGS = 256
GROUP_SIZES_K = (256, 256, 256, 192)

NC, NS = 2, 16
NW = NC * NS
TOK_PER_W = B // NW
CH = 8
NCH = TOK_PER_W // CH
NBUF = 3

TILE = 512
NT = B // TILE


def _mesh():
    return plsc.VectorSubcoreMesh(core_axis_name="c", subcore_axis_name="s")


def _sc_permute_body(hid3, src2, dest3, hs_out, idx_v, dst_v, buf0, buf1,
                     buf2, sem_g, sem_s):
    wid = lax.axis_index("s") * NC + lax.axis_index("c")
    pltpu.sync_copy(src2.at[wid], idx_v)
    pltpu.sync_copy(dest3.at[wid], dst_v)
    bufs = (buf0, buf1, buf2)
    lane = lax.broadcasted_iota(jnp.int32, (16,), 0)

    def start_batch(c, buf):
        vec = idx_v[pl.ds((c // 2) * 16, 16)]
        cps = []
        for j in range(CH):
            l = (c % 2) * CH + j
            s = jnp.max(jnp.where(lane == l, vec, jnp.int32(-1)))
            tok = lax.shift_right_logical(s, 2)
            grp = s - lax.shift_left(tok, 2)
            cps.append(pltpu.async_copy(hid3.at[tok, grp], buf.at[j], sem_g))
        return cps

    pending = [start_batch(0, bufs[0]), start_batch(1, bufs[1])]
    for c in range(NCH):
        cur = bufs[c % NBUF]
        for cp in pending.pop(0):
            cp.wait()
        if c + 2 < NCH:
            pending.append(start_batch(c + 2, bufs[(c + 2) % NBUF]))
        pltpu.async_copy(cur, hs_out.at[dst_v.at[c]], sem_s).wait()


def _sc_unpermute_body(psorted, dest, preds_out, didx_v, pbuf0, pbuf1,
                       sem1, sem2):
    wid = lax.axis_index("s") * NC + lax.axis_index("c")
    base = wid * TOK_PER_W
    half = TOK_PER_W // 2
    pltpu.sync_copy(dest.at[pl.ds(base, TOK_PER_W)], didx_v)
    cp1 = pltpu.async_copy(psorted.at[didx_v.at[pl.ds(0, half)]], pbuf0, sem1)
    cp2 = pltpu.async_copy(psorted.at[didx_v.at[pl.ds(half, half)]], pbuf1,
                           sem2)
    cp1.wait()
    pltpu.sync_copy(pbuf0, preds_out.at[pl.ds(base, half)])
    cp2.wait()
    pltpu.sync_copy(pbuf1, preds_out.at[pl.ds(base + half, half)])


def _tc_gmm_body(h_ref, gs_ref, go_ref, w0, w1, w2, w3, b0, b1, b2, b3,
                 out_ref, valid_ref):
    gs_row = gs_ref[0, 0, :]
    gmin = jnp.min(gs_row)
    gmax = jnp.max(gs_row)
    hb = h_ref[...].astype(jnp.bfloat16)
    for gi, (wr, br) in enumerate(zip((w0, w1, w2, w3), (b0, b1, b2, b3))):
        @pl.when((gi >= gmin) & (gi <= gmax))
        def _(wr=wr, br=br, gi=gi):
            pmat = lax.dot_general(hb, wr[...], (((1,), (1,)), ((), ())),
                                   preferred_element_type=jnp.float32)
            pmat = pmat + br[...]
            mask = gs_row[:, None] == gi
            if gi == 0:
                out_ref[...] = jnp.where(mask, pmat, jnp.zeros_like(pmat))
            else:
                @pl.when(gi == gmin)
                def _():
                    out_ref[...] = jnp.where(mask, pmat, jnp.zeros_like(pmat))
                @pl.when(gi != gmin)
                def _():
                    out_ref[...] = jnp.where(mask, pmat, out_ref[...])
    go_row = go_ref[0, 0, :]
    gsz = jnp.full_like(go_row, GROUP_SIZES_K[0])
    for kk in range(1, NG):
        gsz = jnp.where(go_row == kk, GROUP_SIZES_K[kk], gsz)
    cols = lax.broadcasted_iota(jnp.int32, (TILE, MAXGS), 1)
    valid_ref[...] = (cols < gsz[:, None]).astype(jnp.int8)


def _sc_permute(hid3, src2, dest3):
    fn = pl.kernel(
        _sc_permute_body,
        out_type=jax.ShapeDtypeStruct((B, D), jnp.float32),
        mesh=_mesh(),
        compiler_params=pltpu.CompilerParams(needs_layout_passes=False),
        scratch_types=[
            pltpu.VMEM((TOK_PER_W,), jnp.int32),
            pltpu.VMEM((NCH, CH), jnp.int32),
            pltpu.VMEM((CH, D), jnp.float32),
            pltpu.VMEM((CH, D), jnp.float32),
            pltpu.VMEM((CH, D), jnp.float32),
            pltpu.SemaphoreType.DMA,
            pltpu.SemaphoreType.DMA,
        ],
    )
    return fn(hid3, src2, dest3)


def _sc_unpermute(psorted, dest):
    fn = pl.kernel(
        _sc_unpermute_body,
        out_type=jax.ShapeDtypeStruct((B, MAXGS), jnp.float32),
        mesh=_mesh(),
        scratch_types=[
            pltpu.VMEM((TOK_PER_W,), jnp.int32),
            pltpu.VMEM((TOK_PER_W // 2, MAXGS), jnp.float32),
            pltpu.VMEM((TOK_PER_W // 2, MAXGS), jnp.float32),
            pltpu.SemaphoreType.DMA,
            pltpu.SemaphoreType.DMA,
        ],
    )
    return fn(psorted, dest)


def _tc_gmm(h_sorted, g_sorted3, g_orig3, ws, bs):
    return pl.pallas_call(
        _tc_gmm_body,
        grid=(NT,),
        in_specs=[
            pl.BlockSpec((TILE, D), lambda t: (t, 0)),
            pl.BlockSpec((1, 1, TILE), lambda t: (t, 0, 0)),
            pl.BlockSpec((1, 1, TILE), lambda t: (t, 0, 0)),
            pl.BlockSpec((MAXGS, D), lambda t: (0, 0)),
            pl.BlockSpec((MAXGS, D), lambda t: (0, 0)),
            pl.BlockSpec((MAXGS, D), lambda t: (0, 0)),
            pl.BlockSpec((MAXGS, D), lambda t: (0, 0)),
            pl.BlockSpec((1, MAXGS), lambda t: (0, 0)),
            pl.BlockSpec((1, MAXGS), lambda t: (0, 0)),
            pl.BlockSpec((1, MAXGS), lambda t: (0, 0)),
            pl.BlockSpec((1, MAXGS), lambda t: (0, 0)),
        ],
        out_specs=(pl.BlockSpec((TILE, MAXGS), lambda t: (t, 0)),
                   pl.BlockSpec((TILE, MAXGS), lambda t: (t, 0))),
        out_shape=(jax.ShapeDtypeStruct((B, MAXGS), jnp.float32),
                   jax.ShapeDtypeStruct((B, MAXGS), jnp.int8)),
    )(h_sorted, g_sorted3, g_orig3, *ws, *bs)


def kernel(hidden, chosen_group_idx, W0, W1, W2, W3, b0, b1, b2, b3):
    g = chosen_group_idx.astype(jnp.int32)

    onehot = (g[:, None] == jnp.arange(NG, dtype=jnp.int32)[None, :])
    onehot = onehot.astype(jnp.int32)
    counts = jnp.sum(onehot, axis=0)
    offsets = jnp.cumsum(counts) - counts
    rank = jnp.cumsum(onehot, axis=0) - onehot
    dest = jnp.sum((rank + offsets[None, :]) * onehot, axis=1)
    src_row = NG * jnp.arange(B, dtype=jnp.int32) + g

    p = jnp.arange(B, dtype=jnp.int32)
    g_sorted = ((p >= offsets[1]).astype(jnp.int32) +
                (p >= offsets[2]).astype(jnp.int32) +
                (p >= offsets[3]).astype(jnp.int32))

    h_sorted = _sc_permute(hidden, src_row.reshape(NW, TOK_PER_W),
                           dest.reshape(NW, NCH, CH))

    ws = (W0.astype(jnp.bfloat16),
          W1.astype(jnp.bfloat16),
          W2.astype(jnp.bfloat16),
          jnp.pad(W3.astype(jnp.bfloat16), ((0, MAXGS - GROUP_SIZES_K[3]),
                                            (0, 0))))
    bs = (b0.reshape(1, MAXGS), b1.reshape(1, MAXGS), b2.reshape(1, MAXGS),
          jnp.pad(b3, (0, MAXGS - GROUP_SIZES_K[3])).reshape(1, MAXGS))
    psorted, valid_i8 = _tc_gmm(h_sorted, g_sorted.reshape(NT, 1, TILE),
                                g.reshape(NT, 1, TILE), ws, bs)

    preds = _sc_unpermute(psorted, dest)
    return preds, valid_i8.astype(jnp.bool_)
